# baseline (speedup 1.0000x reference)
import jax
import jax.numpy as jnp
from jax.experimental import pallas as pl
from jax.experimental.pallas import tpu as pltpu

_BM = 512
_BN = 256
_INV_SQRT2 = 0.7071067811865476


def _matmul_gelu_kernel(a_ref, w_ref, b_ref, o_ref, w_bf16):
    mi = pl.program_id(1)

    @pl.when(mi == 0)
    def _convert():
        w_bf16[...] = w_ref[...].astype(jnp.bfloat16)

    bm = o_ref.shape[0]
    a = a_ref[pl.ds(mi * bm, bm), :]
    acc = jnp.dot(a, w_bf16[...], preferred_element_type=jnp.float32)
    x = acc + b_ref[...]
    o_ref[...] = x * (0.5 * (1.0 + jax.lax.erf(x * _INV_SQRT2)))


def kernel(hidden_states, W, b):
    batch, seq, d_in = hidden_states.shape
    m = batch * seq
    k_dim, n = W.shape
    a = hidden_states.reshape(m, d_in).astype(jnp.bfloat16)
    b2 = b.reshape(1, n)

    bm, bn = min(_BM, m), min(_BN, n)
    grid = (n // bn, m // bm)

    out = pl.pallas_call(
        _matmul_gelu_kernel,
        grid=grid,
        in_specs=[
            pl.BlockSpec((m, k_dim), lambda ni, mi: (0, 0)),
            pl.BlockSpec((k_dim, bn), lambda ni, mi: (0, ni)),
            pl.BlockSpec((1, bn), lambda ni, mi: (0, ni)),
        ],
        out_specs=pl.BlockSpec((bm, bn), lambda ni, mi: (mi, ni)),
        out_shape=jax.ShapeDtypeStruct((m, n), jnp.float32),
        scratch_shapes=[pltpu.VMEM((k_dim, bn), jnp.bfloat16)],
        compiler_params=pltpu.CompilerParams(
            dimension_semantics=("parallel", "arbitrary"),
        ),
    )(a, W, b2)
    return out.reshape(batch, seq, n)


# speedup vs baseline: 1.1927x; 1.1927x over previous
import functools

import jax
import jax.numpy as jnp
from jax.experimental import pallas as pl
from jax.experimental.pallas import tpu as pltpu

_BM, _BN, _BK = 2048, 2048, 512
_INV_SQRT2 = 0.7071067811865476


def _matmul_gelu_kernel(a_ref, w_ref, b_ref, o_ref, *, k_steps):
    k = pl.program_id(2)

    @pl.when(k == 0)
    def _init():
        o_ref[...] = jnp.broadcast_to(b_ref[...], o_ref.shape)

    o_ref[...] += jnp.dot(a_ref[...], w_ref[...],
                          preferred_element_type=jnp.float32)

    @pl.when(k == k_steps - 1)
    def _finish():
        x = o_ref[...]
        o_ref[...] = x * (0.5 * (1.0 + jax.lax.erf(x * _INV_SQRT2)))


def kernel(hidden_states, W, b):
    batch, seq, d_in = hidden_states.shape
    m = batch * seq
    k_dim, n = W.shape
    a = hidden_states.reshape(m, d_in)
    b2 = b.reshape(1, n)

    bm, bn, bk = min(_BM, m), min(_BN, n), min(_BK, k_dim)
    k_steps = k_dim // bk
    grid = (m // bm, n // bn, k_steps)

    out = pl.pallas_call(
        functools.partial(_matmul_gelu_kernel, k_steps=k_steps),
        grid=grid,
        in_specs=[
            pl.BlockSpec((bm, bk), lambda mi, ni, ki: (mi, ki)),
            pl.BlockSpec((bk, bn), lambda mi, ni, ki: (ki, ni)),
            pl.BlockSpec((1, bn), lambda mi, ni, ki: (0, ni)),
        ],
        out_specs=pl.BlockSpec((bm, bn), lambda mi, ni, ki: (mi, ni)),
        out_shape=jax.ShapeDtypeStruct((m, n), jnp.float32),
        compiler_params=pltpu.CompilerParams(
            dimension_semantics=("parallel", "parallel", "arbitrary"),
        ),
    )(a, W, b2)
    return out.reshape(batch, seq, n)
